# Initial kernel scaffold; baseline (speedup 1.0000x reference)
import jax
import jax.numpy as jnp
from jax.experimental import pallas as pl


def kernel(x, edge_index, edge_attr, batch, W_init, b_init, W1, b1, W2, b2, W_ffn, b_ffn):
    raise NotImplementedError("write your pallas kernel here")


# speedup vs baseline: 2.7058x; 2.7058x over previous
import functools

import jax
import jax.numpy as jnp
from jax import lax
from jax.experimental import pallas as pl
from jax.experimental.pallas import tpu as pltpu
from jax.experimental.pallas import tpu_sc as plsc

_N, _E, _F, _H, _DEPTH, _G = 10000, 320000, 128, 128, 4, 64
_NC, _NS, _L = 2, 16, 16
_NW = _NC * _NS
_CH = 64
_NJ = 160
_NB = 4
_IBJ = _NJ // _NB
_SLAB = _NJ * _CH
_ROWS_PT = 640
_N_PAD = _NS * _ROWS_PT
_BLK = 2000
_STASH0 = 10048
_NSTASH = 2 * _NS
_SH_SIZES = [10080] * 11 + [9840] * 4 + [9760]

def _make_sc_segment_sum():
    mesh = plsc.VectorSubcoreMesh(core_axis_name="c", subcore_axis_name="s")
    return functools.partial(
        pl.kernel,
        out_type=jax.ShapeDtypeStruct((_NC, _N_PAD, _H), jnp.float32),
        mesh=mesh,
        scratch_types=[
            pltpu.VMEM((_IBJ, _CH), jnp.int32),
            pltpu.VMEM((_IBJ, _CH), jnp.int32),
            pltpu.VMEM((_CH, _H), jnp.float32),
            pltpu.VMEM((_CH, _H), jnp.float32),
            pltpu.VMEM_SHARED((_N_PAD, _H), jnp.float32),
            pltpu.SemaphoreType.DMA,
            pltpu.SemaphoreType.DMA,
        ],
    )(_sc_segment_sum_body)


def _sc_segment_sum_body(h_hbm, src_hbm, dst_hbm, out_hbm,
                         src_v, dst_v, buf0, buf1, agg_sh, sem0, sem1):
    c = lax.axis_index("c")
    s = lax.axis_index("s")
    w = c * _NS + s

    @pl.loop(0, _CH)
    def _(r):
        @pl.loop(0, _H, step=_L)
        def _(col):
            buf0[r, pl.ds(col, _L)] = jnp.zeros((_L,), jnp.float32)

    @pl.loop(0, _ROWS_PT, step=_CH)
    def _(r0):
        pltpu.sync_copy(buf0, agg_sh.at[pl.ds(s * _ROWS_PT + r0, _CH)])

    plsc.subcore_barrier()

    @pl.loop(0, _NB)
    def _(blk):
        pltpu.sync_copy(src_hbm.at[w, pl.ds(blk * _IBJ, _IBJ)], src_v)
        pltpu.sync_copy(dst_hbm.at[w, pl.ds(blk * _IBJ, _IBJ)], dst_v)
        pltpu.async_copy(h_hbm.at[src_v.at[0]], buf0, sem0)

        @pl.loop(0, _IBJ, step=2)
        def _(j):
            pltpu.async_copy(h_hbm.at[src_v.at[j + 1]], buf1, sem1)
            pltpu.make_async_copy(h_hbm.at[src_v.at[j]], buf0, sem0).wait()
            pltpu.sync_copy(buf0, agg_sh.at[dst_v.at[j]], add=True)

            @pl.when(j + 2 < _IBJ)
            def _():
                pltpu.async_copy(h_hbm.at[src_v.at[j + 2]], buf0, sem0)

            pltpu.make_async_copy(h_hbm.at[src_v.at[j + 1]], buf1, sem1).wait()
            pltpu.sync_copy(buf1, agg_sh.at[dst_v.at[j + 1]], add=True)

    plsc.subcore_barrier()
    pltpu.sync_copy(agg_sh.at[pl.ds(s * _ROWS_PT, _ROWS_PT)],
                    out_hbm.at[c].at[pl.ds(s * _ROWS_PT, _ROWS_PT)])


def _dot(a, b, dims=None, precision=lax.Precision.DEFAULT):
    if dims is None:
        return jnp.dot(a, b, preferred_element_type=jnp.float32,
                       precision=precision)
    return lax.dot_general(a, b, (dims, ((), ())),
                           preferred_element_type=jnp.float32,
                           precision=precision)


def _init_body(x_ref, w_ref, b_ref, o_ref):
    o_ref[...] = jnp.maximum(_dot(x_ref[...], w_ref[...]) + b_ref[...], 0.0)


_mlp_init = pl.pallas_call(
    _init_body,
    grid=(_N // _BLK,),
    in_specs=[pl.BlockSpec((_BLK, _F), lambda i: (i, 0)),
              pl.BlockSpec((_F, _H), lambda i: (0, 0)),
              pl.BlockSpec((1, _H), lambda i: (0, 0))],
    out_specs=pl.BlockSpec((_BLK, _H), lambda i: (i, 0)),
    out_shape=jax.ShapeDtypeStruct((_N, _H), jnp.float32),
)


def _layer_body(p_ref, stash_ref, ids_ref, h_ref, h0_ref,
                w1_ref, b1_ref, w2_ref, b2_ref, o_ref, agg_ref):
    i = pl.program_id(0)
    agg_ref[...] = p_ref[0] + p_ref[1]
    for k in range(2 * _NSTASH):
        loc = ids_ref[k] - i * _BLK

        @pl.when((loc >= 0) & (loc < _BLK))
        def _():
            agg_ref[pl.ds(loc, 1), :] += stash_ref[k // _NSTASH,
                                                   pl.ds(k % _NSTASH, 1), :]

    m = agg_ref[...] + h_ref[...]
    t = jnp.maximum(_dot(m, w1_ref[...]) + b1_ref[...], 0.0)
    u = _dot(t, w2_ref[...]) + b2_ref[...]
    o_ref[...] = jnp.maximum(u + h0_ref[...], 0.0)


_gin_layer = pl.pallas_call(
    _layer_body,
    grid=(_N // _BLK,),
    in_specs=[pl.BlockSpec((_NC, _BLK, _H), lambda i: (0, i, 0)),
              pl.BlockSpec((_NC, _NSTASH, _H),
                           lambda i: (0, _STASH0 // _NSTASH, 0)),
              pl.BlockSpec(memory_space=pltpu.SMEM),
              pl.BlockSpec((_BLK, _H), lambda i: (i, 0)),
              pl.BlockSpec((_BLK, _H), lambda i: (i, 0)),
              pl.BlockSpec((_H, _H), lambda i: (0, 0)),
              pl.BlockSpec((1, _H), lambda i: (0, 0)),
              pl.BlockSpec((_H, _H), lambda i: (0, 0)),
              pl.BlockSpec((1, _H), lambda i: (0, 0))],
    out_specs=pl.BlockSpec((_BLK, _H), lambda i: (i, 0)),
    out_shape=jax.ShapeDtypeStruct((_N, _H), jnp.float32),
    scratch_shapes=[pltpu.VMEM((_BLK, _H), jnp.float32)],
)


def _pool_body(h_ref, batch_ref, wf_ref, bf_ref, o_ref, acc_ref):
    i = pl.program_id(0)

    @pl.when(i == 0)
    def _():
        acc_ref[...] = jnp.zeros_like(acc_ref)

    onehot = (batch_ref[...] ==
              lax.broadcasted_iota(jnp.int32, (1, _H), 1)).astype(jnp.float32)
    acc_ref[...] += _dot(h_ref[...], onehot, ((0,), (0,)),
                         precision=lax.Precision.HIGHEST)

    @pl.when(i == pl.num_programs(0) - 1)
    def _():
        o_ref[...] = _dot(wf_ref[...], acc_ref[...], ((0,), (0,))) + bf_ref[...]


_pool = pl.pallas_call(
    _pool_body,
    grid=(_N // _BLK,),
    in_specs=[pl.BlockSpec((_BLK, _H), lambda i: (i, 0)),
              pl.BlockSpec((_BLK, 1), lambda i: (i, 0)),
              pl.BlockSpec((_H, 1), lambda i: (0, 0)),
              pl.BlockSpec((1, 1), lambda i: (0, 0))],
    out_specs=pl.BlockSpec((1, _H), lambda i: (0, 0)),
    out_shape=jax.ShapeDtypeStruct((1, _H), jnp.float32),
    scratch_shapes=[pltpu.VMEM((_H, _H), jnp.float32)],
)


_W_SIZES = _SH_SIZES + _SH_SIZES
_W_STARTS = []
for _c in range(_NC):
    _off = _c * (_E // _NC)
    for _sz in _SH_SIZES:
        _W_STARTS.append(_off)
        _off += _sz


def kernel(x, edge_index, edge_attr, batch,
           W_init, b_init, W1, b1, W2, b2, W_ffn, b_ffn):
    src = edge_index[0]
    dst = edge_index[1]
    sdst, ssrc = lax.sort([dst, src], num_keys=1, is_stable=True)

    starts = jnp.asarray(_W_STARTS, jnp.int32)
    sizes = jnp.asarray(_W_SIZES, jnp.int32)
    pos = starts[:, None] + jnp.arange(_SLAB, dtype=jnp.int32)[None, :]
    valid = jnp.arange(_SLAB, dtype=jnp.int32)[None, :] < sizes[:, None]
    posc = jnp.clip(pos, 0, _E - 1)
    dsl = jnp.where(valid, sdst[posc], _N)
    ssl = jnp.where(valid, ssrc[posc], 0)
    lead = dsl[:, 0]
    trail = sdst[starts + sizes - 1]
    srow = jnp.asarray([_STASH0 + 2 * (w % _NS) for w in range(_NW)],
                       jnp.int32)
    dred = jnp.where(dsl == lead[:, None], srow[:, None],
                     jnp.where(dsl == trail[:, None], srow[:, None] + 1, dsl))
    srcp = ssl.reshape(_NW, _NJ, _CH)
    dstp = dred.reshape(_NW, _NJ, _CH)
    ids = jnp.stack([lead, trail], axis=1).reshape(-1)

    sc_segment_sum = _make_sc_segment_sum()
    h0 = _mlp_init(x, W_init, b_init.reshape(1, _H))
    h = h0
    for i in range(_DEPTH):
        p = sc_segment_sum(h, srcp, dstp)
        h = _gin_layer(p, p, ids, h, h0, W1[i], b1[i].reshape(1, _H),
                       W2[i], b2[i].reshape(1, _H))
    res = _pool(h, batch.reshape(_N, 1), W_ffn, b_ffn.reshape(1, 1))
    return res[0, :_G]


# speedup vs baseline: 2.7745x; 1.0254x over previous
import functools

import jax
import jax.numpy as jnp
from jax import lax
from jax.experimental import pallas as pl
from jax.experimental.pallas import tpu as pltpu
from jax.experimental.pallas import tpu_sc as plsc

_N, _E, _F, _H, _DEPTH, _G = 10000, 320000, 128, 128, 4, 64
_NC, _NS, _L = 2, 16, 16
_NW = _NC * _NS
_CH = 128
_NJ = 80
_NB = 5
_IBJ = _NJ // _NB
_SLAB = _NJ * _CH
_ROWS_PT = 632
_N_PAD = _NS * _ROWS_PT
_BLK = 2000
_STASH0 = 10048
_NSTASH = 2 * _NS
_SH_SIZES = [10080] * 11 + [9840] * 4 + [9760]

def _make_sc_segment_sum():
    mesh = plsc.VectorSubcoreMesh(core_axis_name="c", subcore_axis_name="s")
    return functools.partial(
        pl.kernel,
        out_type=jax.ShapeDtypeStruct((_NC, _N_PAD, _H), jnp.float32),
        mesh=mesh,
        scratch_types=[
            pltpu.VMEM((_IBJ, _CH), jnp.int32),
            pltpu.VMEM((_IBJ, _CH), jnp.int32),
            pltpu.VMEM((_CH, _H), jnp.float32),
            pltpu.VMEM((_CH, _H), jnp.float32),
            pltpu.VMEM_SHARED((_N_PAD, _H), jnp.float32),
            pltpu.SemaphoreType.DMA,
            pltpu.SemaphoreType.DMA,
        ],
    )(_sc_segment_sum_body)


def _sc_segment_sum_body(h_hbm, src_hbm, dst_hbm, out_hbm,
                         src_v, dst_v, buf0, buf1, agg_sh, sem0, sem1):
    c = lax.axis_index("c")
    s = lax.axis_index("s")
    w = c * _NS + s

    @pl.loop(0, _CH)
    def _(r):
        @pl.loop(0, _H, step=_L)
        def _(col):
            buf0[r, pl.ds(col, _L)] = jnp.zeros((_L,), jnp.float32)

    @pl.loop(0, _ROWS_PT - _CH, step=_CH)
    def _(r0):
        pltpu.sync_copy(buf0, agg_sh.at[pl.ds(s * _ROWS_PT + r0, _CH)])

    _rem = _ROWS_PT % _CH
    pltpu.sync_copy(buf0.at[pl.ds(0, _rem)],
                    agg_sh.at[pl.ds(s * _ROWS_PT + _ROWS_PT - _rem, _rem)])

    plsc.subcore_barrier()

    @pl.loop(0, _NB)
    def _(blk):
        pltpu.sync_copy(src_hbm.at[w, pl.ds(blk * _IBJ, _IBJ)], src_v)
        pltpu.sync_copy(dst_hbm.at[w, pl.ds(blk * _IBJ, _IBJ)], dst_v)
        pltpu.async_copy(h_hbm.at[src_v.at[0]], buf0, sem0)

        @pl.loop(0, _IBJ, step=2)
        def _(j):
            pltpu.async_copy(h_hbm.at[src_v.at[j + 1]], buf1, sem1)
            pltpu.make_async_copy(h_hbm.at[src_v.at[j]], buf0, sem0).wait()
            pltpu.sync_copy(buf0, agg_sh.at[dst_v.at[j]], add=True)

            @pl.when(j + 2 < _IBJ)
            def _():
                pltpu.async_copy(h_hbm.at[src_v.at[j + 2]], buf0, sem0)

            pltpu.make_async_copy(h_hbm.at[src_v.at[j + 1]], buf1, sem1).wait()
            pltpu.sync_copy(buf1, agg_sh.at[dst_v.at[j + 1]], add=True)

    plsc.subcore_barrier()
    pltpu.sync_copy(agg_sh.at[pl.ds(s * _ROWS_PT, _ROWS_PT)],
                    out_hbm.at[c].at[pl.ds(s * _ROWS_PT, _ROWS_PT)])


def _dot(a, b, dims=None, precision=lax.Precision.DEFAULT):
    if dims is None:
        return jnp.dot(a, b, preferred_element_type=jnp.float32,
                       precision=precision)
    return lax.dot_general(a, b, (dims, ((), ())),
                           preferred_element_type=jnp.float32,
                           precision=precision)


def _init_body(x_ref, w_ref, b_ref, o_ref):
    o_ref[...] = jnp.maximum(_dot(x_ref[...], w_ref[...]) + b_ref[...], 0.0)


_mlp_init = pl.pallas_call(
    _init_body,
    grid=(_N // _BLK,),
    in_specs=[pl.BlockSpec((_BLK, _F), lambda i: (i, 0)),
              pl.BlockSpec((_F, _H), lambda i: (0, 0)),
              pl.BlockSpec((1, _H), lambda i: (0, 0))],
    out_specs=pl.BlockSpec((_BLK, _H), lambda i: (i, 0)),
    out_shape=jax.ShapeDtypeStruct((_N, _H), jnp.float32),
)


def _layer_body(p_ref, stash_ref, ids_ref, h_ref, h0_ref,
                w1_ref, b1_ref, w2_ref, b2_ref, o_ref, agg_ref):
    i = pl.program_id(0)
    agg_ref[...] = p_ref[0] + p_ref[1]
    for k in range(2 * _NSTASH):
        loc = ids_ref[k] - i * _BLK

        @pl.when((loc >= 0) & (loc < _BLK))
        def _():
            agg_ref[pl.ds(loc, 1), :] += stash_ref[k // _NSTASH,
                                                   pl.ds(k % _NSTASH, 1), :]

    m = agg_ref[...] + h_ref[...]
    t = jnp.maximum(_dot(m, w1_ref[...]) + b1_ref[...], 0.0)
    u = _dot(t, w2_ref[...]) + b2_ref[...]
    o_ref[...] = jnp.maximum(u + h0_ref[...], 0.0)


_gin_layer = pl.pallas_call(
    _layer_body,
    grid=(_N // _BLK,),
    in_specs=[pl.BlockSpec((_NC, _BLK, _H), lambda i: (0, i, 0)),
              pl.BlockSpec((_NC, _NSTASH, _H),
                           lambda i: (0, _STASH0 // _NSTASH, 0)),
              pl.BlockSpec(memory_space=pltpu.SMEM),
              pl.BlockSpec((_BLK, _H), lambda i: (i, 0)),
              pl.BlockSpec((_BLK, _H), lambda i: (i, 0)),
              pl.BlockSpec((_H, _H), lambda i: (0, 0)),
              pl.BlockSpec((1, _H), lambda i: (0, 0)),
              pl.BlockSpec((_H, _H), lambda i: (0, 0)),
              pl.BlockSpec((1, _H), lambda i: (0, 0))],
    out_specs=pl.BlockSpec((_BLK, _H), lambda i: (i, 0)),
    out_shape=jax.ShapeDtypeStruct((_N, _H), jnp.float32),
    scratch_shapes=[pltpu.VMEM((_BLK, _H), jnp.float32)],
)


def _pool_body(h_ref, batch_ref, wf_ref, bf_ref, o_ref, acc_ref):
    i = pl.program_id(0)

    @pl.when(i == 0)
    def _():
        acc_ref[...] = jnp.zeros_like(acc_ref)

    onehot = (batch_ref[...] ==
              lax.broadcasted_iota(jnp.int32, (1, _H), 1)).astype(jnp.float32)
    acc_ref[...] += _dot(h_ref[...], onehot, ((0,), (0,)),
                         precision=lax.Precision.HIGHEST)

    @pl.when(i == pl.num_programs(0) - 1)
    def _():
        o_ref[...] = _dot(wf_ref[...], acc_ref[...], ((0,), (0,))) + bf_ref[...]


_pool = pl.pallas_call(
    _pool_body,
    grid=(_N // _BLK,),
    in_specs=[pl.BlockSpec((_BLK, _H), lambda i: (i, 0)),
              pl.BlockSpec((_BLK, 1), lambda i: (i, 0)),
              pl.BlockSpec((_H, 1), lambda i: (0, 0)),
              pl.BlockSpec((1, 1), lambda i: (0, 0))],
    out_specs=pl.BlockSpec((1, _H), lambda i: (0, 0)),
    out_shape=jax.ShapeDtypeStruct((1, _H), jnp.float32),
    scratch_shapes=[pltpu.VMEM((_H, _H), jnp.float32)],
)


_W_SIZES = _SH_SIZES + _SH_SIZES
_W_STARTS = []
for _c in range(_NC):
    _off = _c * (_E // _NC)
    for _sz in _SH_SIZES:
        _W_STARTS.append(_off)
        _off += _sz


def kernel(x, edge_index, edge_attr, batch,
           W_init, b_init, W1, b1, W2, b2, W_ffn, b_ffn):
    src = edge_index[0]
    dst = edge_index[1]
    sdst, ssrc = lax.sort([dst, src], num_keys=1, is_stable=True)

    starts = jnp.asarray(_W_STARTS, jnp.int32)
    sizes = jnp.asarray(_W_SIZES, jnp.int32)
    pos = starts[:, None] + jnp.arange(_SLAB, dtype=jnp.int32)[None, :]
    valid = jnp.arange(_SLAB, dtype=jnp.int32)[None, :] < sizes[:, None]
    posc = jnp.clip(pos, 0, _E - 1)
    dsl = jnp.where(valid, sdst[posc], _N)
    ssl = jnp.where(valid, ssrc[posc], 0)
    lead = dsl[:, 0]
    trail = sdst[starts + sizes - 1]
    srow = jnp.asarray([_STASH0 + 2 * (w % _NS) for w in range(_NW)],
                       jnp.int32)
    dred = jnp.where(dsl == lead[:, None], srow[:, None],
                     jnp.where(dsl == trail[:, None], srow[:, None] + 1, dsl))
    srcp = ssl.reshape(_NW, _NJ, _CH)
    dstp = dred.reshape(_NW, _NJ, _CH)
    ids = jnp.stack([lead, trail], axis=1).reshape(-1)

    sc_segment_sum = _make_sc_segment_sum()
    h0 = _mlp_init(x, W_init, b_init.reshape(1, _H))
    h = h0
    for i in range(_DEPTH):
        p = sc_segment_sum(h, srcp, dstp)
        h = _gin_layer(p, p, ids, h, h0, W1[i], b1[i].reshape(1, _H),
                       W2[i], b2[i].reshape(1, _H))
    res = _pool(h, batch.reshape(_N, 1), W_ffn, b_ffn.reshape(1, 1))
    return res[0, :_G]
